# baseline (speedup 1.0000x reference)
import functools

import jax
import jax.numpy as jnp
from jax import lax
from jax.experimental import pallas as pl
from jax.experimental.pallas import tpu as pltpu
from jax.experimental.pallas import tpu_sc as plsc

_N = 10000
_E = 320000
_D = 128
_K = 20
_NLAYERS = 2

_NPAD = 10240
_NHALF = _NPAD // 2
_CHUNK = 256
_NCH = 42
_G = 6
_NGRP = _NCH // _G
_EPW = _NCH * _CHUNK
_EPSC = 16 * _EPW
_RPT = _NHALF // 16
_NBUF = 2
_LOOK = 1
_W = 1


@functools.cache
def _build_sc_prop():
  mesh = plsc.VectorSubcoreMesh(core_axis_name="c", subcore_axis_name="s")

  @functools.partial(
      pl.kernel,
      out_type=jax.ShapeDtypeStruct((_NPAD, _D), jnp.float32),
      mesh=mesh,
      scratch_types=[
          pltpu.VMEM((2, _G, _CHUNK), jnp.int32),
          pltpu.VMEM((2, _G, _CHUNK), jnp.int32),
          pltpu.VMEM((_NBUF, _CHUNK, _D), jnp.float32),
          pltpu.VMEM((16, _D), jnp.float32),
          pltpu.VMEM_SHARED((_NHALF, _D), jnp.float32),
          pltpu.SemaphoreType.DMA,
          pltpu.SemaphoreType.DMA,
      ] + [pltpu.SemaphoreType.DMA] * (2 * _NBUF),
      compiler_params=pltpu.CompilerParams(use_tc_tiling_on_sc=False),
  )
  def _sc_prop_impl(z_hbm, g_hbm, s_hbm, out_hbm, gv, sv, rows, zb, acc,
                    sem_i, sem_z, *bsems):
    cid = lax.axis_index("c")
    sid = lax.axis_index("s")
    sems_g = bsems[:_NBUF]
    sems_s = bsems[_NBUF:]
    base = sid * _RPT

    def issue_gather(si, j, b):
        pltpu.async_copy(z_hbm.at[gv.at[si, j]], rows.at[b], sems_g[b])

    def wait_gather(si, j, b):
        pltpu.make_async_copy(z_hbm.at[gv.at[si, j]], rows.at[b],
                              sems_g[b]).wait()

    def issue_scatter(si, j, b):
        pltpu.async_copy(rows.at[b], acc.at[sv.at[si, j]], sems_s[b], add=True)

    def wait_scatter(si, j, b):
        pltpu.make_async_copy(rows.at[b], acc.at[sv.at[si, j]],
                              sems_s[b]).wait()

    pltpu.async_copy(g_hbm.at[cid, sid, 0], gv.at[0], sem_i)
    pltpu.async_copy(s_hbm.at[cid, sid, 0], sv.at[0], sem_i)

    @pl.loop(0, 16)
    def _zrow(i):
        @pl.loop(0, _D, step=16)
        def _zcol(j):
            zb[i, pl.ds(j, 16)] = jnp.zeros((16,), jnp.float32)

    pltpu.make_async_copy(g_hbm.at[cid, sid, 0], gv.at[0], sem_i).wait()
    pltpu.make_async_copy(s_hbm.at[cid, sid, 0], sv.at[0], sem_i).wait()

    for b in range(_LOOK):
        issue_gather(0, b, b)

    @pl.loop(0, _RPT, step=16)
    def _zacc(r):
        pltpu.async_copy(zb, acc.at[pl.ds(base + r, 16)], sem_z).wait()

    plsc.subcore_barrier()

    @pl.loop(0, _NGRP)
    def _grp(gi):
        si = lax.rem(gi, 2)
        osi = 1 - si
        for j in range(_G):
            c = gi * _G + j
            b = j % _NBUF
            wait_gather(si, j, b)
            issue_scatter(si, j, b)

            jw = j - _W
            bw = jw % _NBUF
            @pl.when(c >= _W)
            def _():
                if jw >= 0:
                    wait_scatter(si, jw, bw)
                else:
                    wait_scatter(osi, jw + _G, bw)

            if j == _W:
                @pl.when(gi + 1 < _NGRP)
                def _():
                    pltpu.async_copy(g_hbm.at[cid, sid, gi + 1], gv.at[osi],
                                     sem_i)
                    pltpu.async_copy(s_hbm.at[cid, sid, gi + 1], sv.at[osi],
                                     sem_i)

            if j == _G - _LOOK:
                @pl.when(gi + 1 < _NGRP)
                def _():
                    pltpu.make_async_copy(g_hbm.at[cid, sid, gi + 1],
                                          gv.at[osi], sem_i).wait()
                    pltpu.make_async_copy(s_hbm.at[cid, sid, gi + 1],
                                          sv.at[osi], sem_i).wait()

            ji = j + _LOOK
            bi = ji % _NBUF
            if ji < _G:
                issue_gather(si, ji, bi)
            else:
                @pl.when(gi + 1 < _NGRP)
                def _():
                    issue_gather(osi, ji - _G, bi)

    lsi = (_NGRP - 1) % 2
    for c in range(_NCH - _W, _NCH):
        wait_scatter(lsi, c - (_NGRP - 1) * _G, c % _NBUF)
    plsc.subcore_barrier()

    pltpu.sync_copy(acc.at[pl.ds(base, _RPT)],
                    out_hbm.at[pl.ds(cid * _NHALF + base, _RPT)])

  return _sc_prop_impl


def _sc_prop(z, g2, s2):
    return _build_sc_prop()(z, g2, s2)


def _partition(key, gidx, sidx):
    gs, ss = [], []
    spread_g = _N + (jnp.arange(_EPSC, dtype=jnp.int32) % (_NPAD - _N))
    spread_s = jnp.arange(_EPSC, dtype=jnp.int32) % _NHALF
    for c in range(2):
        inhalf = (key >= c * _NHALF) & (key < (c + 1) * _NHALF)
        sel = jnp.nonzero(inhalf, size=_EPSC, fill_value=_E)[0]
        ispad = sel >= _E
        gsafe = jnp.minimum(sel, _E - 1)
        g = jnp.where(ispad, spread_g, gidx[gsafe])
        s = jnp.where(ispad, spread_s, sidx[gsafe] - c * _NHALF)
        gs.append(g.reshape(16, _NGRP, _G, _CHUNK))
        ss.append(s.reshape(16, _NGRP, _G, _CHUNK))
    return jnp.stack(gs), jnp.stack(ss)


def _row_mask(nrows):
    return lax.broadcasted_iota(jnp.int32, (nrows, 1), 0) < _N


def _bn_relu(t, g, b):
    mask = _row_mask(t.shape[0])
    tm = jnp.where(mask, t, 0.0)
    mu = jnp.sum(tm, axis=0, keepdims=True) * (1.0 / _N)
    d = jnp.where(mask, t - mu, 0.0)
    var = jnp.sum(d * d, axis=0, keepdims=True) * (1.0 / _N)
    return jax.nn.relu((t - mu) * lax.rsqrt(var + 1e-5) * g + b)


def _mlp(x, w_ref, b_ref, g_ref, bb_ref):
    h = jnp.dot(x, w_ref[0], preferred_element_type=jnp.float32) + b_ref[0]
    h = _bn_relu(h, g_ref[0], bb_ref[0])
    h = jnp.dot(h, w_ref[1], preferred_element_type=jnp.float32) + b_ref[1]
    h = jnp.dot(h, w_ref[2], preferred_element_type=jnp.float32) + b_ref[2]
    h = _bn_relu(h, g_ref[1], bb_ref[1])
    return jnp.dot(h, w_ref[3], preferred_element_type=jnp.float32) + b_ref[3]


def _enc_body(x_ref, w_ref, b_ref, g_ref, bb_ref, degp_ref, w0_ref,
              h_ref, z_ref, dis_ref, out0_ref):
    mask = _row_mask(_NPAD)
    h = _mlp(x_ref[...], w_ref, b_ref, g_ref, bb_ref)
    h = jnp.where(mask, h, 0.0)
    deg = degp_ref[:, 0:1]
    dis = jnp.where(deg > 0, lax.rsqrt(jnp.maximum(deg, 1e-12)), 0.0)
    dis = jnp.where(mask, dis, 0.0)
    h_ref[...] = h
    z_ref[...] = dis * h
    dis_ref[...] = dis
    out0_ref[...] = jnp.dot(h, w0_ref[...], preferred_element_type=jnp.float32)


_enc_call = pl.pallas_call(
    _enc_body,
    out_shape=[
        jax.ShapeDtypeStruct((_NPAD, _D), jnp.float32),
        jax.ShapeDtypeStruct((_NPAD, _D), jnp.float32),
        jax.ShapeDtypeStruct((_NPAD, 1), jnp.float32),
        jax.ShapeDtypeStruct((_NPAD, _D), jnp.float32),
    ],
)


def _make_step(first):
    def body(p_ref, dis_ref, *rest):
        if first:
            w_ref, outin_ref, tx_ref, z_ref, out_ref = rest
            t = (-dis_ref[...]) * p_ref[...]
        else:
            tx0_ref, w_ref, outin_ref, tx_ref, z_ref, out_ref = rest
            t = (-2.0 * dis_ref[...]) * p_ref[...] - tx0_ref[...]
        tx_ref[...] = t
        z_ref[...] = dis_ref[...] * t
        out_ref[...] = outin_ref[...] + jnp.dot(
            t, w_ref[...], preferred_element_type=jnp.float32)

    blk = 1024
    in_specs = [
        pl.BlockSpec((blk, _D), lambda i: (i, 0)),
        pl.BlockSpec((blk, 1), lambda i: (i, 0)),
    ]
    if not first:
        in_specs.append(pl.BlockSpec((blk, _D), lambda i: (i, 0)))
    in_specs += [
        pl.BlockSpec((_D, _D), lambda i: (0, 0)),
        pl.BlockSpec((blk, _D), lambda i: (i, 0)),
    ]
    nin = len(in_specs)
    return pl.pallas_call(
        body,
        grid=(_NPAD // blk,),
        in_specs=in_specs,
        out_specs=[pl.BlockSpec((blk, _D), lambda i: (i, 0))] * 3,
        out_shape=[jax.ShapeDtypeStruct((_NPAD, _D), jnp.float32)] * 3,
        input_output_aliases={nin - 1: 2},
    )


_step_first = _make_step(True)
_step_rest = _make_step(False)


def _fin_body(out_ref, cb_ref, g_ref, bb_ref, dis_ref, w0_ref,
              h_ref, z_ref, out0_ref):
    mask = _row_mask(_NPAD)
    h = _bn_relu(out_ref[...] + cb_ref[...], g_ref[...], bb_ref[...])
    h = jnp.where(mask, h, 0.0)
    dis = dis_ref[...]
    h_ref[...] = h
    z_ref[...] = dis * h
    out0_ref[...] = jnp.dot(h, w0_ref[...], preferred_element_type=jnp.float32)


_fin_call = pl.pallas_call(
    _fin_body,
    out_shape=[
        jax.ShapeDtypeStruct((_NPAD, _D), jnp.float32),
        jax.ShapeDtypeStruct((_NPAD, _D), jnp.float32),
        jax.ShapeDtypeStruct((_NPAD, _D), jnp.float32),
    ],
)


def _dec_body(out_ref, cb_ref, g_ref, bb_ref, w_ref, b_ref, dg_ref, dbb_ref,
              o_ref):
    mask = _row_mask(_NPAD)
    h = _bn_relu(out_ref[...] + cb_ref[...], g_ref[...], bb_ref[...])
    h = jnp.where(mask, h, 0.0)
    y = _mlp(h, w_ref, b_ref, dg_ref, dbb_ref)
    o_ref[...] = y[0:_N, :]


_dec_call = pl.pallas_call(
    _dec_body,
    out_shape=jax.ShapeDtypeStruct((_N, _D), jnp.float32),
)


def kernel(x, edge_index, enc_W, enc_b, enc_bn_g, enc_bn_b, cheb_W, cheb_b,
           gnn_bn_g, gnn_bn_b, dec_W, dec_b, dec_bn_g, dec_bn_b):
    src = edge_index[0].astype(jnp.int32)
    dst = edge_index[1].astype(jnp.int32)
    g2, s2 = _partition(dst, src, dst)
    dg2, ds2 = _partition(src, src, src)
    x_pad = jnp.pad(x, ((0, _NPAD - _N), (0, 0)))
    ones_feat = jnp.where(_row_mask(_NPAD),
                          jnp.float32(1.0), jnp.float32(0.0)) * jnp.ones(
        (_NPAD, _D), dtype=jnp.float32)

    degp = _sc_prop(ones_feat, dg2, ds2)

    h, z, dis, out = _enc_call(x_pad, enc_W, enc_b, enc_bn_g, enc_bn_b, degp,
                               cheb_W[0, 0])

    for layer in range(_NLAYERS):
        tx_prev = h
        p = _sc_prop(z, g2, s2)
        tx_cur, z, out = _step_first(p, dis, cheb_W[layer, 1], out)
        for k in range(2, _K):
            p = _sc_prop(z, g2, s2)
            tx_next, z, out = _step_rest(p, dis, tx_prev, cheb_W[layer, k], out)
            tx_prev, tx_cur = tx_cur, tx_next
        if layer + 1 < _NLAYERS:
            h, z, out = _fin_call(out, cheb_b[layer], gnn_bn_g[layer],
                                  gnn_bn_b[layer], dis, cheb_W[layer + 1, 0])

    return _dec_call(out, cheb_b[_NLAYERS - 1], gnn_bn_g[_NLAYERS - 1],
                     gnn_bn_b[_NLAYERS - 1], dec_W, dec_b, dec_bn_g, dec_bn_b)


# speedup vs baseline: 1.1247x; 1.1247x over previous
import functools

import jax
import jax.numpy as jnp
from jax import lax
from jax.experimental import pallas as pl
from jax.experimental.pallas import tpu as pltpu
from jax.experimental.pallas import tpu_sc as plsc

_N = 10000
_E = 320000
_D = 128
_K = 20
_NLAYERS = 2

_NPAD = 10240
_NHALF = _NPAD // 2
_CHUNK = 128
_NCH = 84
_G = 12
_NGRP = _NCH // _G
_EPW = _NCH * _CHUNK
_EPSC = 16 * _EPW
_RPT = _NHALF // 16
_NBUF = 4
_LOOK = 3
_W = 1


@functools.cache
def _build_sc_prop():
  mesh = plsc.VectorSubcoreMesh(core_axis_name="c", subcore_axis_name="s")

  @functools.partial(
      pl.kernel,
      out_type=jax.ShapeDtypeStruct((_NPAD, _D), jnp.float32),
      mesh=mesh,
      scratch_types=[
          pltpu.VMEM((2, _G, _CHUNK), jnp.int32),
          pltpu.VMEM((2, _G, _CHUNK), jnp.int32),
          pltpu.VMEM((_NBUF, _CHUNK, _D), jnp.float32),
          pltpu.VMEM((16, _D), jnp.float32),
          pltpu.VMEM_SHARED((_NHALF, _D), jnp.float32),
          pltpu.SemaphoreType.DMA,
          pltpu.SemaphoreType.DMA,
      ] + [pltpu.SemaphoreType.DMA] * (2 * _NBUF),
      compiler_params=pltpu.CompilerParams(use_tc_tiling_on_sc=False),
  )
  def _sc_prop_impl(z_hbm, g_hbm, s_hbm, out_hbm, gv, sv, rows, zb, acc,
                    sem_i, sem_z, *bsems):
    cid = lax.axis_index("c")
    sid = lax.axis_index("s")
    sems_g = bsems[:_NBUF]
    sems_s = bsems[_NBUF:]
    base = sid * _RPT

    def issue_gather(si, j, b):
        pltpu.async_copy(z_hbm.at[gv.at[si, j]], rows.at[b], sems_g[b])

    def wait_gather(si, j, b):
        pltpu.make_async_copy(z_hbm.at[gv.at[si, j]], rows.at[b],
                              sems_g[b]).wait()

    def issue_scatter(si, j, b):
        pltpu.async_copy(rows.at[b], acc.at[sv.at[si, j]], sems_s[b], add=True)

    def wait_scatter(si, j, b):
        pltpu.make_async_copy(rows.at[b], acc.at[sv.at[si, j]],
                              sems_s[b]).wait()

    pltpu.async_copy(g_hbm.at[cid, sid, 0], gv.at[0], sem_i)
    pltpu.async_copy(s_hbm.at[cid, sid, 0], sv.at[0], sem_i)

    @pl.loop(0, 16)
    def _zrow(i):
        @pl.loop(0, _D, step=16)
        def _zcol(j):
            zb[i, pl.ds(j, 16)] = jnp.zeros((16,), jnp.float32)

    pltpu.make_async_copy(g_hbm.at[cid, sid, 0], gv.at[0], sem_i).wait()
    pltpu.make_async_copy(s_hbm.at[cid, sid, 0], sv.at[0], sem_i).wait()

    for b in range(_LOOK):
        issue_gather(0, b, b)

    @pl.loop(0, _RPT, step=16)
    def _zacc(r):
        pltpu.async_copy(zb, acc.at[pl.ds(base + r, 16)], sem_z).wait()

    plsc.subcore_barrier()

    @pl.loop(0, _NGRP)
    def _grp(gi):
        si = lax.rem(gi, 2)
        osi = 1 - si
        for j in range(_G):
            c = gi * _G + j
            b = j % _NBUF
            wait_gather(si, j, b)
            issue_scatter(si, j, b)

            jw = j - _W
            bw = jw % _NBUF
            @pl.when(c >= _W)
            def _():
                if jw >= 0:
                    wait_scatter(si, jw, bw)
                else:
                    wait_scatter(osi, jw + _G, bw)

            if j == _W:
                @pl.when(gi + 1 < _NGRP)
                def _():
                    pltpu.async_copy(g_hbm.at[cid, sid, gi + 1], gv.at[osi],
                                     sem_i)
                    pltpu.async_copy(s_hbm.at[cid, sid, gi + 1], sv.at[osi],
                                     sem_i)

            if j == _G - _LOOK:
                @pl.when(gi + 1 < _NGRP)
                def _():
                    pltpu.make_async_copy(g_hbm.at[cid, sid, gi + 1],
                                          gv.at[osi], sem_i).wait()
                    pltpu.make_async_copy(s_hbm.at[cid, sid, gi + 1],
                                          sv.at[osi], sem_i).wait()

            ji = j + _LOOK
            bi = ji % _NBUF
            if ji < _G:
                issue_gather(si, ji, bi)
            else:
                @pl.when(gi + 1 < _NGRP)
                def _():
                    issue_gather(osi, ji - _G, bi)

    lsi = (_NGRP - 1) % 2
    for c in range(_NCH - _W, _NCH):
        wait_scatter(lsi, c - (_NGRP - 1) * _G, c % _NBUF)
    plsc.subcore_barrier()

    pltpu.sync_copy(acc.at[pl.ds(base, _RPT)],
                    out_hbm.at[pl.ds(cid * _NHALF + base, _RPT)])

  return _sc_prop_impl


def _sc_prop(z, g2, s2):
    return _build_sc_prop()(z, g2, s2)


def _partition(key, gidx, sidx):
    gs, ss = [], []
    spread_g = _N + (jnp.arange(_EPSC, dtype=jnp.int32) % (_NPAD - _N))
    spread_s = jnp.arange(_EPSC, dtype=jnp.int32) % _NHALF
    for c in range(2):
        inhalf = (key >= c * _NHALF) & (key < (c + 1) * _NHALF)
        sel = jnp.nonzero(inhalf, size=_EPSC, fill_value=_E)[0]
        ispad = sel >= _E
        gsafe = jnp.minimum(sel, _E - 1)
        g = jnp.where(ispad, spread_g, gidx[gsafe])
        s = jnp.where(ispad, spread_s, sidx[gsafe] - c * _NHALF)
        gs.append(g.reshape(16, _NGRP, _G, _CHUNK))
        ss.append(s.reshape(16, _NGRP, _G, _CHUNK))
    return jnp.stack(gs), jnp.stack(ss)


def _row_mask(nrows):
    return lax.broadcasted_iota(jnp.int32, (nrows, 1), 0) < _N


def _bn_relu(t, g, b):
    mask = _row_mask(t.shape[0])
    tm = jnp.where(mask, t, 0.0)
    mu = jnp.sum(tm, axis=0, keepdims=True) * (1.0 / _N)
    d = jnp.where(mask, t - mu, 0.0)
    var = jnp.sum(d * d, axis=0, keepdims=True) * (1.0 / _N)
    return jax.nn.relu((t - mu) * lax.rsqrt(var + 1e-5) * g + b)


def _mlp(x, w_ref, b_ref, g_ref, bb_ref):
    h = jnp.dot(x, w_ref[0], preferred_element_type=jnp.float32) + b_ref[0]
    h = _bn_relu(h, g_ref[0], bb_ref[0])
    h = jnp.dot(h, w_ref[1], preferred_element_type=jnp.float32) + b_ref[1]
    h = jnp.dot(h, w_ref[2], preferred_element_type=jnp.float32) + b_ref[2]
    h = _bn_relu(h, g_ref[1], bb_ref[1])
    return jnp.dot(h, w_ref[3], preferred_element_type=jnp.float32) + b_ref[3]


def _enc_body(x_ref, w_ref, b_ref, g_ref, bb_ref, degp_ref, w0_ref,
              h_ref, z_ref, dis_ref, out0_ref):
    mask = _row_mask(_NPAD)
    h = _mlp(x_ref[...], w_ref, b_ref, g_ref, bb_ref)
    h = jnp.where(mask, h, 0.0)
    deg = degp_ref[:, 0:1]
    dis = jnp.where(deg > 0, lax.rsqrt(jnp.maximum(deg, 1e-12)), 0.0)
    dis = jnp.where(mask, dis, 0.0)
    h_ref[...] = h
    z_ref[...] = dis * h
    dis_ref[...] = dis
    out0_ref[...] = jnp.dot(h, w0_ref[...], preferred_element_type=jnp.float32)


_enc_call = pl.pallas_call(
    _enc_body,
    out_shape=[
        jax.ShapeDtypeStruct((_NPAD, _D), jnp.float32),
        jax.ShapeDtypeStruct((_NPAD, _D), jnp.float32),
        jax.ShapeDtypeStruct((_NPAD, 1), jnp.float32),
        jax.ShapeDtypeStruct((_NPAD, _D), jnp.float32),
    ],
)


def _make_step(first):
    def body(p_ref, dis_ref, *rest):
        if first:
            w_ref, outin_ref, tx_ref, z_ref, out_ref = rest
            t = (-dis_ref[...]) * p_ref[...]
        else:
            tx0_ref, w_ref, outin_ref, tx_ref, z_ref, out_ref = rest
            t = (-2.0 * dis_ref[...]) * p_ref[...] - tx0_ref[...]
        tx_ref[...] = t
        z_ref[...] = dis_ref[...] * t
        out_ref[...] = outin_ref[...] + jnp.dot(
            t, w_ref[...], preferred_element_type=jnp.float32)

    blk = 1024
    in_specs = [
        pl.BlockSpec((blk, _D), lambda i: (i, 0)),
        pl.BlockSpec((blk, 1), lambda i: (i, 0)),
    ]
    if not first:
        in_specs.append(pl.BlockSpec((blk, _D), lambda i: (i, 0)))
    in_specs += [
        pl.BlockSpec((_D, _D), lambda i: (0, 0)),
        pl.BlockSpec((blk, _D), lambda i: (i, 0)),
    ]
    nin = len(in_specs)
    return pl.pallas_call(
        body,
        grid=(_NPAD // blk,),
        in_specs=in_specs,
        out_specs=[pl.BlockSpec((blk, _D), lambda i: (i, 0))] * 3,
        out_shape=[jax.ShapeDtypeStruct((_NPAD, _D), jnp.float32)] * 3,
        input_output_aliases={nin - 1: 2},
    )


_step_first = _make_step(True)
_step_rest = _make_step(False)


def _fin_body(out_ref, cb_ref, g_ref, bb_ref, dis_ref, w0_ref,
              h_ref, z_ref, out0_ref):
    mask = _row_mask(_NPAD)
    h = _bn_relu(out_ref[...] + cb_ref[...], g_ref[...], bb_ref[...])
    h = jnp.where(mask, h, 0.0)
    dis = dis_ref[...]
    h_ref[...] = h
    z_ref[...] = dis * h
    out0_ref[...] = jnp.dot(h, w0_ref[...], preferred_element_type=jnp.float32)


_fin_call = pl.pallas_call(
    _fin_body,
    out_shape=[
        jax.ShapeDtypeStruct((_NPAD, _D), jnp.float32),
        jax.ShapeDtypeStruct((_NPAD, _D), jnp.float32),
        jax.ShapeDtypeStruct((_NPAD, _D), jnp.float32),
    ],
)


def _dec_body(out_ref, cb_ref, g_ref, bb_ref, w_ref, b_ref, dg_ref, dbb_ref,
              o_ref):
    mask = _row_mask(_NPAD)
    h = _bn_relu(out_ref[...] + cb_ref[...], g_ref[...], bb_ref[...])
    h = jnp.where(mask, h, 0.0)
    y = _mlp(h, w_ref, b_ref, dg_ref, dbb_ref)
    o_ref[...] = y[0:_N, :]


_dec_call = pl.pallas_call(
    _dec_body,
    out_shape=jax.ShapeDtypeStruct((_N, _D), jnp.float32),
)


def kernel(x, edge_index, enc_W, enc_b, enc_bn_g, enc_bn_b, cheb_W, cheb_b,
           gnn_bn_g, gnn_bn_b, dec_W, dec_b, dec_bn_g, dec_bn_b):
    src = edge_index[0].astype(jnp.int32)
    dst = edge_index[1].astype(jnp.int32)
    g2, s2 = _partition(dst, src, dst)
    dg2, ds2 = _partition(src, src, src)
    x_pad = jnp.pad(x, ((0, _NPAD - _N), (0, 0)))
    ones_feat = jnp.where(_row_mask(_NPAD),
                          jnp.float32(1.0), jnp.float32(0.0)) * jnp.ones(
        (_NPAD, _D), dtype=jnp.float32)

    degp = _sc_prop(ones_feat, dg2, ds2)

    h, z, dis, out = _enc_call(x_pad, enc_W, enc_b, enc_bn_g, enc_bn_b, degp,
                               cheb_W[0, 0])

    for layer in range(_NLAYERS):
        tx_prev = h
        p = _sc_prop(z, g2, s2)
        tx_cur, z, out = _step_first(p, dis, cheb_W[layer, 1], out)
        for k in range(2, _K):
            p = _sc_prop(z, g2, s2)
            tx_next, z, out = _step_rest(p, dis, tx_prev, cheb_W[layer, k], out)
            tx_prev, tx_cur = tx_cur, tx_next
        if layer + 1 < _NLAYERS:
            h, z, out = _fin_call(out, cheb_b[layer], gnn_bn_g[layer],
                                  gnn_bn_b[layer], dis, cheb_W[layer + 1, 0])

    return _dec_call(out, cheb_b[_NLAYERS - 1], gnn_bn_g[_NLAYERS - 1],
                     gnn_bn_b[_NLAYERS - 1], dec_W, dec_b, dec_bn_g, dec_bn_b)
